# baseline (speedup 1.0000x reference)
import functools

import jax
import jax.numpy as jnp
from jax import lax
from jax.experimental import pallas as pl
from jax.experimental.pallas import tpu as pltpu
from jax.experimental.pallas import tpu_sc as plsc

_N_CLASSES = 1000
_LANES = 16
_ZWORDS_PER_TILE = 10_000


@functools.cache
def _make_onehot(n_rows, n_classes, seq_len):
    info = plsc.get_sparse_core_info()
    nc, ns = info.num_cores, info.num_subcores
    n_workers = nc * ns
    rows_per_w = n_rows // n_workers
    out_words = n_rows * n_classes
    words_per_sc = out_words // nc
    zwords = _ZWORDS_PER_TILE * ns
    dmas_per_sc = words_per_sc // zwords
    dmas_per_tile = dmas_per_sc // ns
    assert words_per_sc % zwords == 0 and dmas_per_sc % ns == 0
    assert _ZWORDS_PER_TILE % _LANES == 0 and rows_per_w % _LANES == 0
    mesh = plsc.VectorSubcoreMesh(core_axis_name="c", subcore_axis_name="s")

    l_per_sc = seq_len // nc
    b_per_tile = n_rows // seq_len // ns
    magic = (1 << 17) // l_per_sc + 1
    assert all((i * magic) >> 17 == i // l_per_sc for i in range(rows_per_w))

    @functools.partial(
        pl.kernel,
        out_type=jax.ShapeDtypeStruct((out_words,), jnp.float32),
        mesh=mesh,
        scratch_types=[
            pltpu.VMEM((_ZWORDS_PER_TILE,), jnp.float32),
            pltpu.VMEM_SHARED((zwords,), jnp.float32),
            pltpu.VMEM((n_rows,), jnp.int32),
            pltpu.VMEM((rows_per_w,), jnp.int32),
            pltpu.VMEM((rows_per_w,), jnp.float32),
            pltpu.SemaphoreType.DMA,
            pltpu.SemaphoreType.DMA,
        ],
        compiler_params=pltpu.CompilerParams(needs_layout_passes=False),
    )
    def k(x_hbm, out_hbm, zb, zshared, x_v, idx_v, ones_v, zsem, ssem):
        c = lax.axis_index("c")
        s = lax.axis_index("s")

        zeros16 = jnp.zeros((_LANES,), jnp.float32)
        ones16 = jnp.ones((_LANES,), jnp.float32)
        iota16 = lax.iota(jnp.int32, _LANES)

        def zero_body(i, carry):
            zb[pl.ds(i * _LANES, _LANES)] = zeros16
            return carry

        lax.fori_loop(0, _ZWORDS_PER_TILE // _LANES, zero_body, 0)
        pltpu.sync_copy(zb, zshared.at[pl.ds(s * _ZWORDS_PER_TILE,
                                             _ZWORDS_PER_TILE)])

        pltpu.sync_copy(x_hbm, x_v)

        l0 = c * l_per_sc
        b0 = s * b_per_tile

        def idx_body(i, carry):
            flat = i * _LANES + iota16
            bloc = (flat * magic) >> 17
            lloc = flat - bloc * l_per_sc
            b = b0 + bloc
            l = l0 + lloc
            cls = plsc.load_gather(x_v, [b * seq_len + l])
            off = (l * (n_classes * 1024)
                   + (cls >> 3) * 8192
                   + (b >> 7) * 1024
                   + (cls & 7) * 128
                   + (b & 127))
            idx_v[pl.ds(i * _LANES, _LANES)] = off
            ones_v[pl.ds(i * _LANES, _LANES)] = ones16
            return carry

        lax.fori_loop(0, rows_per_w // _LANES, idx_body, 0)

        plsc.subcore_barrier()

        sc_base = c * words_per_sc

        def fire_body(j, carry):
            dst0 = sc_base + (s * dmas_per_tile + j) * zwords
            pltpu.async_copy(zshared, out_hbm.at[pl.ds(dst0, zwords)], zsem)
            return carry

        lax.fori_loop(0, dmas_per_tile, fire_body, 0)

        def drain_body(j, carry):
            pltpu.make_async_copy(
                zshared, out_hbm.at[pl.ds(0, zwords)], zsem).wait()
            return carry

        lax.fori_loop(0, dmas_per_tile, drain_body, 0)

        plsc.subcore_barrier()

        pltpu.async_copy(ones_v, out_hbm.at[idx_v], ssem).wait()

    return k


def kernel(x):
    b, l = x.shape
    n_rows = b * l
    xf = x.reshape(n_rows).astype(jnp.int32)
    out1d = _make_onehot(n_rows, _N_CLASSES, l)(xf)
    out5 = out1d.reshape(l, _N_CLASSES // 8, b // 128, 8, 128)
    outt = jnp.transpose(out5, (2, 4, 0, 1, 3))
    return outt.reshape(b, l, _N_CLASSES)


# speedup vs baseline: 1.3141x; 1.3141x over previous
import functools

import jax
import jax.numpy as jnp
from jax import lax
from jax.experimental import pallas as pl
from jax.experimental.pallas import tpu as pltpu
from jax.experimental.pallas import tpu_sc as plsc

_N_CLASSES = 1000
_LANES = 16
_ZCHUNK = 2_048_000


def _zero_body(o_ref):
    o_ref[...] = jnp.zeros((_ZCHUNK,), jnp.float32)


@functools.cache
def _make_zero_fill(out_words):
    assert out_words % _ZCHUNK == 0
    return pl.pallas_call(
        _zero_body,
        grid=(out_words // _ZCHUNK,),
        out_specs=pl.BlockSpec((_ZCHUNK,), lambda i: (i,)),
        out_shape=jax.ShapeDtypeStruct((out_words,), jnp.float32),
    )


@functools.cache
def _make_scatter(n_rows, n_classes, seq_len):
    info = plsc.get_sparse_core_info()
    nc, ns = info.num_cores, info.num_subcores
    rows_per_w = n_rows // (nc * ns)
    l_per_sc = seq_len // nc
    b_per_tile = n_rows // seq_len // ns
    magic = (1 << 17) // l_per_sc + 1
    assert all((i * magic) >> 17 == i // l_per_sc for i in range(rows_per_w))
    assert rows_per_w % _LANES == 0
    mesh = plsc.VectorSubcoreMesh(core_axis_name="c", subcore_axis_name="s")

    @functools.partial(
        pl.kernel,
        out_type=(),
        mesh=mesh,
        scratch_types=[
            pltpu.VMEM((n_rows,), jnp.int32),
            pltpu.VMEM((rows_per_w,), jnp.int32),
            pltpu.VMEM((rows_per_w,), jnp.float32),
            pltpu.SemaphoreType.DMA,
        ],
        compiler_params=pltpu.CompilerParams(needs_layout_passes=False),
    )
    def k(x_hbm, out_hbm, x_v, idx_v, ones_v, ssem):
        c = lax.axis_index("c")
        s = lax.axis_index("s")

        ones16 = jnp.ones((_LANES,), jnp.float32)
        iota16 = lax.iota(jnp.int32, _LANES)

        pltpu.sync_copy(x_hbm, x_v)

        l0 = c * l_per_sc
        b0 = s * b_per_tile

        def idx_body(i, carry):
            flat = i * _LANES + iota16
            bloc = (flat * magic) >> 17
            lloc = flat - bloc * l_per_sc
            b = b0 + bloc
            l = l0 + lloc
            cls = plsc.load_gather(x_v, [b * seq_len + l])
            off = (l * (n_classes * 1024)
                   + (cls >> 3) * 8192
                   + (b >> 7) * 1024
                   + (cls & 7) * 128
                   + (b & 127))
            idx_v[pl.ds(i * _LANES, _LANES)] = off
            ones_v[pl.ds(i * _LANES, _LANES)] = ones16
            return carry

        lax.fori_loop(0, rows_per_w // _LANES, idx_body, 0)

        pltpu.async_copy(ones_v, out_hbm.at[idx_v], ssem).wait()

    return k


def kernel(x):
    b, l = x.shape
    n_rows = b * l
    out_words = n_rows * _N_CLASSES
    xf = x.reshape(n_rows).astype(jnp.int32)
    buf = jax.new_ref(_make_zero_fill(out_words)())
    _make_scatter(n_rows, _N_CLASSES, l)(xf, buf)
    out1d = buf[...]
    out5 = out1d.reshape(l, _N_CLASSES // 8, b // 128, 8, 128)
    outt = jnp.transpose(out5, (2, 4, 0, 1, 3))
    return outt.reshape(b, l, _N_CLASSES)


# speedup vs baseline: 1.3601x; 1.0350x over previous
import functools

import jax
import jax.numpy as jnp
from jax import lax
from jax.experimental import pallas as pl
from jax.experimental.pallas import tpu as pltpu
from jax.experimental.pallas import tpu_sc as plsc

_N_CLASSES = 1000
_LANES = 16
_ZCHUNK = 2_048_000


def _zero_body(o_ref):
    o_ref[...] = jnp.zeros((_ZCHUNK,), jnp.float32)


@functools.cache
def _make_zero_fill(out_words):
    assert out_words % _ZCHUNK == 0
    return pl.pallas_call(
        _zero_body,
        grid=(out_words // _ZCHUNK,),
        out_specs=pl.BlockSpec((_ZCHUNK,), lambda i: (i,)),
        out_shape=jax.ShapeDtypeStruct((out_words,), jnp.float32),
    )


@functools.cache
def _make_scatter(n_rows, n_classes, seq_len):
    info = plsc.get_sparse_core_info()
    nc, ns = info.num_cores, info.num_subcores
    rows_per_w = n_rows // (nc * ns)
    l_per_sc = seq_len // nc
    b_per_tile = n_rows // seq_len // ns
    magic = (1 << 17) // l_per_sc + 1
    assert all((i * magic) >> 17 == i // l_per_sc for i in range(rows_per_w))
    assert rows_per_w % _LANES == 0
    mesh = plsc.VectorSubcoreMesh(core_axis_name="c", subcore_axis_name="s")

    @functools.partial(
        pl.kernel,
        out_type=(),
        mesh=mesh,
        scratch_types=[
            pltpu.VMEM((b_per_tile, seq_len), jnp.int32),
            pltpu.VMEM((rows_per_w,), jnp.int32),
            pltpu.VMEM((rows_per_w,), jnp.float32),
            pltpu.SemaphoreType.DMA,
        ],
        compiler_params=pltpu.CompilerParams(needs_layout_passes=False),
    )
    def k(x_hbm, out_hbm, x_v, idx_v, ones_v, ssem):
        c = lax.axis_index("c")
        s = lax.axis_index("s")

        ones16 = jnp.ones((_LANES,), jnp.float32)
        iota16 = lax.iota(jnp.int32, _LANES)

        l0 = c * l_per_sc
        b0 = s * b_per_tile
        pltpu.sync_copy(x_hbm.at[pl.ds(b0, b_per_tile)], x_v)

        def idx_body(i, carry):
            flat = i * _LANES + iota16
            bloc = (flat * magic) >> 17
            lloc = flat - bloc * l_per_sc
            b = b0 + bloc
            l = l0 + lloc
            cls = plsc.load_gather(x_v, [bloc, l])
            off = (l * (n_classes * 1024)
                   + (cls >> 3) * 8192
                   + (b >> 7) * 1024
                   + (cls & 7) * 128
                   + (b & 127))
            idx_v[pl.ds(i * _LANES, _LANES)] = off
            ones_v[pl.ds(i * _LANES, _LANES)] = ones16
            return carry

        lax.fori_loop(0, rows_per_w // _LANES, idx_body, 0)

        pltpu.async_copy(ones_v, out_hbm.at[idx_v], ssem).wait()

    return k


def kernel(x):
    b, l = x.shape
    n_rows = b * l
    out_words = n_rows * _N_CLASSES
    buf = jax.new_ref(_make_zero_fill(out_words)())
    _make_scatter(n_rows, _N_CLASSES, l)(x.astype(jnp.int32), buf)
    out1d = buf[...]
    out5 = out1d.reshape(l, _N_CLASSES // 8, b // 128, 8, 128)
    outt = jnp.transpose(out5, (2, 4, 0, 1, 3))
    return outt.reshape(b, l, _N_CLASSES)


# speedup vs baseline: 1.3670x; 1.0051x over previous
import functools

import jax
import jax.numpy as jnp
from jax import lax
from jax.experimental import pallas as pl
from jax.experimental.pallas import tpu as pltpu
from jax.experimental.pallas import tpu_sc as plsc

_N_CLASSES = 1000
_LANES = 16
_ZCHUNK = 2_048_000


def _zero_body(o_ref):
    o_ref[...] = jnp.zeros((_ZCHUNK,), jnp.float32)


@functools.cache
def _make_zero_fill(out_words):
    assert out_words % _ZCHUNK == 0
    return pl.pallas_call(
        _zero_body,
        grid=(out_words // _ZCHUNK,),
        out_specs=pl.BlockSpec((_ZCHUNK,), lambda i: (i,)),
        out_shape=jax.ShapeDtypeStruct((out_words,), jnp.float32),
    )


@functools.cache
def _make_scatter(n_rows, n_classes, seq_len):
    info = plsc.get_sparse_core_info()
    nc, ns = info.num_cores, info.num_subcores
    rows_per_w = n_rows // (nc * ns)
    l_per_sc = seq_len // nc
    b_per_tile = n_rows // seq_len // ns
    magic = (1 << 17) // l_per_sc + 1
    assert all((i * magic) >> 17 == i // l_per_sc for i in range(rows_per_w))
    assert rows_per_w % _LANES == 0
    mesh = plsc.VectorSubcoreMesh(core_axis_name="c", subcore_axis_name="s")

    @functools.partial(
        pl.kernel,
        out_type=(),
        mesh=mesh,
        scratch_types=[
            pltpu.VMEM((b_per_tile, seq_len), jnp.int32),
            pltpu.VMEM((rows_per_w,), jnp.int32),
            pltpu.VMEM((rows_per_w,), jnp.float32),
            pltpu.SemaphoreType.DMA,
        ],
        compiler_params=pltpu.CompilerParams(needs_layout_passes=False),
    )
    def k(x_hbm, out_hbm, x_v, idx_v, ones_v, ssem):
        c = lax.axis_index("c")
        s = lax.axis_index("s")

        ones16 = jnp.ones((_LANES,), jnp.float32)
        iota16 = lax.iota(jnp.int32, _LANES)

        l0 = c * l_per_sc
        b0 = s * b_per_tile
        pltpu.sync_copy(x_hbm.at[pl.ds(b0, b_per_tile)], x_v)

        def idx_body(i, carry):
            flat = i * _LANES + iota16
            bloc = (flat * magic) >> 17
            lloc = flat - bloc * l_per_sc
            b = b0 + bloc
            l = l0 + lloc
            cls = plsc.load_gather(x_v, [bloc, l])
            off = (l * (n_classes * 1024)
                   + (cls >> 3) * 8192
                   + (b >> 7) * 1024
                   + (cls & 7) * 128
                   + (b & 127))
            idx_v[pl.ds(i * _LANES, _LANES)] = off
            ones_v[pl.ds(i * _LANES, _LANES)] = ones16
            return carry

        lax.fori_loop(0, rows_per_w // _LANES, idx_body, 0, unroll=4)

        pltpu.async_copy(ones_v, out_hbm.at[idx_v], ssem).wait()

    return k


def kernel(x):
    b, l = x.shape
    n_rows = b * l
    out_words = n_rows * _N_CLASSES
    buf = jax.new_ref(_make_zero_fill(out_words)())
    _make_scatter(n_rows, _N_CLASSES, l)(x.astype(jnp.int32), buf)
    out1d = buf[...]
    out5 = out1d.reshape(l, _N_CLASSES // 8, b // 128, 8, 128)
    outt = jnp.transpose(out5, (2, 4, 0, 1, 3))
    return outt.reshape(b, l, _N_CLASSES)


# speedup vs baseline: 1.3713x; 1.0031x over previous
import functools

import jax
import jax.numpy as jnp
from jax import lax
from jax.experimental import pallas as pl
from jax.experimental.pallas import tpu as pltpu
from jax.experimental.pallas import tpu_sc as plsc

_N_CLASSES = 1000
_LANES = 16
_ZCHUNK = 2_048_000


def _zero_body(o_ref):
    o_ref[...] = jnp.zeros((_ZCHUNK,), jnp.float32)


@functools.cache
def _make_zero_fill(out_words):
    assert out_words % _ZCHUNK == 0
    return pl.pallas_call(
        _zero_body,
        grid=(out_words // _ZCHUNK,),
        out_specs=pl.BlockSpec((_ZCHUNK,), lambda i: (i,)),
        out_shape=jax.ShapeDtypeStruct((out_words,), jnp.float32),
    )


@functools.cache
def _make_scatter(n_rows, n_classes, seq_len):
    info = plsc.get_sparse_core_info()
    nc, ns = info.num_cores, info.num_subcores
    rows_per_w = n_rows // (nc * ns)
    l_per_sc = seq_len // nc
    b_per_tile = n_rows // seq_len // ns
    magic = (1 << 17) // l_per_sc + 1
    assert all((i * magic) >> 17 == i // l_per_sc for i in range(rows_per_w))
    assert rows_per_w % _LANES == 0
    mesh = plsc.VectorSubcoreMesh(core_axis_name="c", subcore_axis_name="s")

    @functools.partial(
        pl.kernel,
        out_type=(),
        mesh=mesh,
        scratch_types=[
            pltpu.VMEM((b_per_tile, seq_len), jnp.int32),
            pltpu.VMEM((rows_per_w,), jnp.int32),
            pltpu.VMEM((rows_per_w,), jnp.float32),
            pltpu.SemaphoreType.DMA,
            pltpu.SemaphoreType.DMA,
        ],
        compiler_params=pltpu.CompilerParams(needs_layout_passes=False),
    )
    def k(x_hbm, out_hbm, x_v, idx_v, ones_v, ssem, ssem2):
        c = lax.axis_index("c")
        s = lax.axis_index("s")

        ones16 = jnp.ones((_LANES,), jnp.float32)
        iota16 = lax.iota(jnp.int32, _LANES)

        l0 = c * l_per_sc
        b0 = s * b_per_tile
        pltpu.sync_copy(x_hbm.at[pl.ds(b0, b_per_tile)], x_v)

        def idx_body(i, carry):
            flat = i * _LANES + iota16
            bloc = (flat * magic) >> 17
            lloc = flat - bloc * l_per_sc
            b = b0 + bloc
            l = l0 + lloc
            cls = plsc.load_gather(x_v, [bloc, l])
            off = (l * (n_classes * 1024)
                   + (cls >> 3) * 8192
                   + (b >> 7) * 1024
                   + (cls & 7) * 128
                   + (b & 127))
            idx_v[pl.ds(i * _LANES, _LANES)] = off
            ones_v[pl.ds(i * _LANES, _LANES)] = ones16
            return carry

        lax.fori_loop(0, rows_per_w // _LANES, idx_body, 0, unroll=4)

        half = rows_per_w // 2
        c1 = pltpu.async_copy(
            ones_v.at[pl.ds(0, half)], out_hbm.at[idx_v.at[pl.ds(0, half)]],
            ssem)
        c2 = pltpu.async_copy(
            ones_v.at[pl.ds(half, half)],
            out_hbm.at[idx_v.at[pl.ds(half, half)]], ssem2)
        c1.wait()
        c2.wait()

    return k


def kernel(x):
    b, l = x.shape
    n_rows = b * l
    out_words = n_rows * _N_CLASSES
    buf = jax.new_ref(_make_zero_fill(out_words)())
    _make_scatter(n_rows, _N_CLASSES, l)(x.astype(jnp.int32), buf)
    out1d = buf[...]
    out5 = out1d.reshape(l, _N_CLASSES // 8, b // 128, 8, 128)
    outt = jnp.transpose(out5, (2, 4, 0, 1, 3))
    return outt.reshape(b, l, _N_CLASSES)
